# baseline (speedup 1.0000x reference)
import functools

import jax
import jax.numpy as jnp
from jax.experimental import pallas as pl
from jax.experimental.pallas import tpu as pltpu
from jax.experimental.pallas import tpu_sc as plsc

_N = 16384
_M = 4096
_D = 64
_K = 3
_BLK = 256
_CBLK = 512

_NC = 2
_NS = 16
_NW = _NC * _NS
_BPW = (_K * _N) // _NW
_CH = 256


def _select_kernel(p_ref, sx_ref, idx_ref, w_ref):
    p = p_ref[...]
    sx = sx_ref[...]
    nn2 = (p[:, 0:1] * p[:, 0:1]
           + p[:, 1:2] * p[:, 1:2]
           + p[:, 2:3] * p[:, 2:3])
    mm2 = (sx[0:1, :] * sx[0:1, :]
           + sx[1:2, :] * sx[1:2, :]
           + sx[2:3, :] * sx[2:3, :])
    dot = jax.lax.dot(p.astype(jnp.bfloat16), sx.astype(jnp.bfloat16),
                      preferred_element_type=jnp.float32)
    d2 = jnp.maximum(nn2 + mm2 - 2.0 * dot, 0.0)

    n_ch = _M // 128
    chunks = [d2[:, j * 128:(j + 1) * 128] for j in range(n_ch)]
    cm = jnp.concatenate(
        [jnp.min(c, axis=1, keepdims=True) for c in chunks], axis=1)

    c_iota = jax.lax.broadcasted_iota(jnp.int32, (_BLK, n_ch), 1)
    cmd = cm
    cjs = []
    for k in range(_K):
        cv = jnp.min(cmd, axis=1, keepdims=True)
        ceq = cmd == cv
        cj = jnp.min(jnp.where(ceq, c_iota, n_ch), axis=1, keepdims=True)
        cjs.append(cj)
        if k < _K - 1:
            cmd = jnp.where(c_iota == cj, jnp.inf, cmd)

    lane = jax.lax.broadcasted_iota(jnp.int32, (_BLK, 128), 1)
    cands = []
    gis = []
    for k in range(_K):
        t = chunks[0]
        for j in range(1, n_ch):
            t = jnp.where(cjs[k] == j, chunks[j], t)
        cands.append(t)
        gis.append(cjs[k] * 128 + lane)

    recips = []
    idxs = []
    for k in range(_K):
        v = jnp.minimum(
            jnp.minimum(jnp.min(cands[0], axis=1, keepdims=True),
                        jnp.min(cands[1], axis=1, keepdims=True)),
            jnp.min(cands[2], axis=1, keepdims=True))
        idx = jnp.minimum(
            jnp.minimum(
                jnp.min(jnp.where(cands[0] == v, gis[0], _M),
                        axis=1, keepdims=True),
                jnp.min(jnp.where(cands[1] == v, gis[1], _M),
                        axis=1, keepdims=True)),
            jnp.min(jnp.where(cands[2] == v, gis[2], _M),
                    axis=1, keepdims=True))
        recips.append(1.0 / (jnp.sqrt(v + 1e-12) + 1e-8))
        idxs.append(idx)
        if k < _K - 1:
            cands = [jnp.where(gis[j] == idx, jnp.inf, cands[j])
                     for j in range(_K)]
    norm = recips[0] + recips[1] + recips[2]
    idx_ref[...] = jnp.where(lane == 0, idxs[0],
                             jnp.where(lane == 1, idxs[1],
                                       jnp.where(lane == 2, idxs[2], 0)))
    w_ref[...] = jnp.where(lane == 0, recips[0] / norm,
                           jnp.where(lane == 1, recips[1] / norm,
                                     jnp.where(lane == 2, recips[2] / norm,
                                               0.0)))


def _sc_gather(s_feat, idx_flat):
    n_idx = idx_flat.shape[0]
    bpw = n_idx // _NW
    mesh = plsc.VectorSubcoreMesh(core_axis_name="c", subcore_axis_name="s")

    @functools.partial(
        pl.kernel, mesh=mesh,
        out_type=jax.ShapeDtypeStruct((n_idx, 128), jnp.float32),
        scratch_types=[pltpu.VMEM((_CH,), jnp.int32),
                       pltpu.VMEM((_CH, 128), jnp.float32),
                       pltpu.SemaphoreType.DMA],
    )
    def k(table_hbm, idx_hbm, out_hbm, idx_v, rows_v, sem):
        wid = jax.lax.axis_index("s") * _NC + jax.lax.axis_index("c")
        base = wid * bpw

        @pl.loop(0, bpw // _CH)
        def _(c):
            b = base + c * _CH
            pltpu.sync_copy(idx_hbm.at[pl.ds(b, _CH)], idx_v)
            pltpu.async_copy(table_hbm.at[idx_v], rows_v, sem).wait()
            pltpu.sync_copy(rows_v, out_hbm.at[pl.ds(b, _CH)])

    return k(s_feat, idx_flat)


def _combine_kernel(g0_ref, g1_ref, g2_ref, w_ref, out_ref):
    w = w_ref[...]
    out_ref[...] = (w[:, 0:1] * g0_ref[:, :_D]
                    + w[:, 1:2] * g1_ref[:, :_D]
                    + w[:, 2:3] * g2_ref[:, :_D])


_H = 2
_NH = _N // _H


def _half_chain(p_half, sx, sf_pad):
    idx128, w128 = pl.pallas_call(
        _select_kernel,
        grid=(_NH // _BLK,),
        in_specs=[
            pl.BlockSpec((_BLK, 128), lambda i: (i, 0)),
            pl.BlockSpec((128, _M), lambda i: (0, 0)),
        ],
        out_specs=[
            pl.BlockSpec((_BLK, 128), lambda i: (i, 0)),
            pl.BlockSpec((_BLK, 128), lambda i: (i, 0)),
        ],
        out_shape=[
            jax.ShapeDtypeStruct((_NH, 128), jnp.int32),
            jax.ShapeDtypeStruct((_NH, 128), jnp.float32),
        ],
        compiler_params=pltpu.CompilerParams(
            dimension_semantics=("parallel",)),
    )(p_half, sx)

    idx_flat = jnp.concatenate(
        [idx128[:, 0], idx128[:, 1], idx128[:, 2]])
    g = _sc_gather(sf_pad, idx_flat)

    nblk = _NH // _CBLK
    return pl.pallas_call(
        _combine_kernel,
        grid=(nblk,),
        in_specs=[
            pl.BlockSpec((_CBLK, 128), lambda i: (i, 0)),
            pl.BlockSpec((_CBLK, 128), lambda i, n=nblk: (i + n, 0)),
            pl.BlockSpec((_CBLK, 128), lambda i, n=nblk: (i + 2 * n, 0)),
            pl.BlockSpec((_CBLK, 128), lambda i: (i, 0)),
        ],
        out_specs=pl.BlockSpec((_CBLK, _D), lambda i: (i, 0)),
        out_shape=jax.ShapeDtypeStruct((_NH, _D), jnp.float32),
        compiler_params=pltpu.CompilerParams(
            dimension_semantics=("parallel",)),
    )(g, g, g, w128)


@jax.jit
def _feat_propagation(parent_coord, s_coord, s_feat):
    p_pad = jnp.zeros((_N, 128), jnp.float32).at[:, :3].set(parent_coord)
    sx = jnp.zeros((128, _M), jnp.float32).at[:3, :].set(s_coord.T)
    sf_pad = jnp.zeros((_M, 128), jnp.float32).at[:, :_D].set(s_feat)
    outs = [_half_chain(p_pad[h * _NH:(h + 1) * _NH], sx, sf_pad)
            for h in range(_H)]
    return jnp.concatenate(outs, axis=0)


def kernel(parent_coord, parent_offset, s_coord, s_offset, s_feat):
    del parent_offset, s_offset
    return _feat_propagation(parent_coord, s_coord, s_feat)


# speedup vs baseline: 1.0631x; 1.0631x over previous
import functools

import jax
import jax.numpy as jnp
from jax.experimental import pallas as pl
from jax.experimental.pallas import tpu as pltpu
from jax.experimental.pallas import tpu_sc as plsc

_N = 16384
_M = 4096
_D = 64
_K = 3
_BLK = 256
_CBLK = 512

_NC = 2
_NS = 16
_NW = _NC * _NS
_BPW = (_K * _N) // _NW
_CH = 256


def _select_kernel(p_ref, sx_ref, idx_ref, w_ref):
    p = p_ref[...]
    sx = sx_ref[...]
    nn2 = (p[:, 0:1] * p[:, 0:1]
           + p[:, 1:2] * p[:, 1:2]
           + p[:, 2:3] * p[:, 2:3])
    mm2 = (sx[0:1, :] * sx[0:1, :]
           + sx[1:2, :] * sx[1:2, :]
           + sx[2:3, :] * sx[2:3, :])
    dot = jax.lax.dot(p.astype(jnp.bfloat16), sx.astype(jnp.bfloat16),
                      preferred_element_type=jnp.float32)
    d2 = jnp.maximum(nn2 + mm2 - 2.0 * dot, 0.0)

    iota = jax.lax.broadcasted_iota(jnp.int32, (_BLK, _M), 1)
    d = d2
    recips = []
    idxs = []
    for k in range(_K):
        v = jnp.min(d, axis=1, keepdims=True)
        eq = d == v
        idx = jnp.min(jnp.where(eq, iota, _M), axis=1, keepdims=True)
        recips.append(1.0 / (jnp.sqrt(v + 1e-12) + 1e-8))
        idxs.append(idx)
        if k < _K - 1:
            d = jnp.where(iota == idx, jnp.inf, d)
    norm = recips[0] + recips[1] + recips[2]
    lane = jax.lax.broadcasted_iota(jnp.int32, (_BLK, 128), 1)
    idx_ref[...] = jnp.where(lane == 0, idxs[0],
                             jnp.where(lane == 1, idxs[1],
                                       jnp.where(lane == 2, idxs[2], 0)))
    w_ref[...] = jnp.where(lane == 0, recips[0] / norm,
                           jnp.where(lane == 1, recips[1] / norm,
                                     jnp.where(lane == 2, recips[2] / norm,
                                               0.0)))


def _sc_gather(s_feat, idx_flat):
    n_idx = idx_flat.shape[0]
    bpw = n_idx // _NW
    ch = _CH if bpw % _CH == 0 else 128
    assert bpw % ch == 0
    mesh = plsc.VectorSubcoreMesh(core_axis_name="c", subcore_axis_name="s")

    @functools.partial(
        pl.kernel, mesh=mesh,
        out_type=jax.ShapeDtypeStruct((n_idx, 128), jnp.float32),
        scratch_types=[pltpu.VMEM((ch,), jnp.int32),
                       pltpu.VMEM((ch, 128), jnp.float32),
                       pltpu.SemaphoreType.DMA],
    )
    def k(table_hbm, idx_hbm, out_hbm, idx_v, rows_v, sem):
        wid = jax.lax.axis_index("s") * _NC + jax.lax.axis_index("c")
        base = wid * bpw

        @pl.loop(0, bpw // ch)
        def _(c):
            b = base + c * ch
            pltpu.sync_copy(idx_hbm.at[pl.ds(b, ch)], idx_v)
            pltpu.async_copy(table_hbm.at[idx_v], rows_v, sem).wait()
            pltpu.sync_copy(rows_v, out_hbm.at[pl.ds(b, ch)])

    return k(s_feat, idx_flat)


def _combine_kernel(g0_ref, g1_ref, g2_ref, w_ref, out_ref):
    w = w_ref[...]
    out_ref[...] = (w[:, 0:1] * g0_ref[:, :_D]
                    + w[:, 1:2] * g1_ref[:, :_D]
                    + w[:, 2:3] * g2_ref[:, :_D])


_H = 4
_NH = _N // _H


def _half_chain(p_half, sx, sf_pad):
    idx128, w128 = pl.pallas_call(
        _select_kernel,
        grid=(_NH // _BLK,),
        in_specs=[
            pl.BlockSpec((_BLK, 128), lambda i: (i, 0)),
            pl.BlockSpec((128, _M), lambda i: (0, 0)),
        ],
        out_specs=[
            pl.BlockSpec((_BLK, 128), lambda i: (i, 0)),
            pl.BlockSpec((_BLK, 128), lambda i: (i, 0)),
        ],
        out_shape=[
            jax.ShapeDtypeStruct((_NH, 128), jnp.int32),
            jax.ShapeDtypeStruct((_NH, 128), jnp.float32),
        ],
        compiler_params=pltpu.CompilerParams(
            dimension_semantics=("parallel",)),
    )(p_half, sx)

    idx_flat = jnp.concatenate(
        [idx128[:, 0], idx128[:, 1], idx128[:, 2]])
    g = _sc_gather(sf_pad, idx_flat)

    nblk = _NH // _CBLK
    return pl.pallas_call(
        _combine_kernel,
        grid=(nblk,),
        in_specs=[
            pl.BlockSpec((_CBLK, 128), lambda i: (i, 0)),
            pl.BlockSpec((_CBLK, 128), lambda i, n=nblk: (i + n, 0)),
            pl.BlockSpec((_CBLK, 128), lambda i, n=nblk: (i + 2 * n, 0)),
            pl.BlockSpec((_CBLK, 128), lambda i: (i, 0)),
        ],
        out_specs=pl.BlockSpec((_CBLK, _D), lambda i: (i, 0)),
        out_shape=jax.ShapeDtypeStruct((_NH, _D), jnp.float32),
        compiler_params=pltpu.CompilerParams(
            dimension_semantics=("parallel",)),
    )(g, g, g, w128)


@jax.jit
def _feat_propagation(parent_coord, s_coord, s_feat):
    p_pad = jnp.zeros((_N, 128), jnp.float32).at[:, :3].set(parent_coord)
    sx = jnp.zeros((128, _M), jnp.float32).at[:3, :].set(s_coord.T)
    sf_pad = jnp.zeros((_M, 128), jnp.float32).at[:, :_D].set(s_feat)
    outs = [_half_chain(p_pad[h * _NH:(h + 1) * _NH], sx, sf_pad)
            for h in range(_H)]
    return jnp.concatenate(outs, axis=0)


def kernel(parent_coord, parent_offset, s_coord, s_offset, s_feat):
    del parent_offset, s_offset
    return _feat_propagation(parent_coord, s_coord, s_feat)


# speedup vs baseline: 1.0966x; 1.0315x over previous
import functools

import jax
import jax.numpy as jnp
from jax.experimental import pallas as pl
from jax.experimental.pallas import tpu as pltpu
from jax.experimental.pallas import tpu_sc as plsc

_N = 16384
_M = 4096
_D = 64
_K = 3
_BLK = 256
_CBLK = 512

_NC = 2
_NS = 16
_NW = _NC * _NS
_BPW = (_K * _N) // _NW
_CH = 256


def _select_kernel(p_ref, sx_ref, idx_ref, w_ref):
    p = p_ref[...]
    sx = sx_ref[...]
    nn2 = (p[:, 0:1] * p[:, 0:1]
           + p[:, 1:2] * p[:, 1:2]
           + p[:, 2:3] * p[:, 2:3])
    mm2 = (sx[0:1, :] * sx[0:1, :]
           + sx[1:2, :] * sx[1:2, :]
           + sx[2:3, :] * sx[2:3, :])
    dot = jax.lax.dot(p.astype(jnp.bfloat16), sx.astype(jnp.bfloat16),
                      preferred_element_type=jnp.float32)
    d2 = jnp.maximum(nn2 + mm2 - 2.0 * dot, 0.0)

    iota = jax.lax.broadcasted_iota(jnp.int32, (_BLK, _M), 1)
    d = d2
    recips = []
    idxs = []
    for k in range(_K):
        v = jnp.min(d, axis=1, keepdims=True)
        eq = d == v
        idx = jnp.min(jnp.where(eq, iota, _M), axis=1, keepdims=True)
        recips.append(1.0 / (jnp.sqrt(v + 1e-12) + 1e-8))
        idxs.append(idx)
        if k < _K - 1:
            d = jnp.where(iota == idx, jnp.inf, d)
    norm = recips[0] + recips[1] + recips[2]
    lane = jax.lax.broadcasted_iota(jnp.int32, (_BLK, 128), 1)
    idx_ref[...] = jnp.where(lane == 0, idxs[0],
                             jnp.where(lane == 1, idxs[1],
                                       jnp.where(lane == 2, idxs[2], 0)))
    w_ref[...] = jnp.where(lane == 0, recips[0] / norm,
                           jnp.where(lane == 1, recips[1] / norm,
                                     jnp.where(lane == 2, recips[2] / norm,
                                               0.0)))


def _sc_gather(s_feat, idx_flat):
    n_idx = idx_flat.shape[0]
    bpw = n_idx // _NW
    ch = _CH if bpw % _CH == 0 else 128
    assert bpw % ch == 0
    mesh = plsc.VectorSubcoreMesh(core_axis_name="c", subcore_axis_name="s")

    @functools.partial(
        pl.kernel, mesh=mesh,
        out_type=jax.ShapeDtypeStruct((n_idx, 128), jnp.float32),
        scratch_types=[pltpu.VMEM((ch,), jnp.int32),
                       pltpu.VMEM((ch, 128), jnp.float32),
                       pltpu.SemaphoreType.DMA],
    )
    def k(table_hbm, idx_hbm, out_hbm, idx_v, rows_v, sem):
        wid = jax.lax.axis_index("s") * _NC + jax.lax.axis_index("c")
        base = wid * bpw

        @pl.loop(0, bpw // ch)
        def _(c):
            b = base + c * ch
            pltpu.sync_copy(idx_hbm.at[pl.ds(b, ch)], idx_v)
            pltpu.async_copy(table_hbm.at[idx_v], rows_v, sem).wait()
            pltpu.sync_copy(rows_v, out_hbm.at[pl.ds(b, ch)])

    return k(s_feat, idx_flat)


def _combine_kernel(g0_ref, g1_ref, g2_ref, w_ref, out_ref):
    w = w_ref[...]
    out_ref[...] = (w[:, 0:1] * g0_ref[:, :_D]
                    + w[:, 1:2] * g1_ref[:, :_D]
                    + w[:, 2:3] * g2_ref[:, :_D])


_H = 2
_NH = _N // _H


def _half_chain(p_half, sx, sf_pad):
    idx128, w128 = pl.pallas_call(
        _select_kernel,
        grid=(_NH // _BLK,),
        in_specs=[
            pl.BlockSpec((_BLK, 128), lambda i: (i, 0)),
            pl.BlockSpec((128, _M), lambda i: (0, 0)),
        ],
        out_specs=[
            pl.BlockSpec((_BLK, 128), lambda i: (i, 0)),
            pl.BlockSpec((_BLK, 128), lambda i: (i, 0)),
        ],
        out_shape=[
            jax.ShapeDtypeStruct((_NH, 128), jnp.int32),
            jax.ShapeDtypeStruct((_NH, 128), jnp.float32),
        ],
        compiler_params=pltpu.CompilerParams(
            dimension_semantics=("parallel",)),
    )(p_half, sx)

    idx_flat = jnp.concatenate(
        [idx128[:, 0], idx128[:, 1], idx128[:, 2]])
    g = _sc_gather(sf_pad, idx_flat)

    nblk = _NH // _CBLK
    return pl.pallas_call(
        _combine_kernel,
        grid=(nblk,),
        in_specs=[
            pl.BlockSpec((_CBLK, 128), lambda i: (i, 0)),
            pl.BlockSpec((_CBLK, 128), lambda i, n=nblk: (i + n, 0)),
            pl.BlockSpec((_CBLK, 128), lambda i, n=nblk: (i + 2 * n, 0)),
            pl.BlockSpec((_CBLK, 128), lambda i: (i, 0)),
        ],
        out_specs=pl.BlockSpec((_CBLK, _D), lambda i: (i, 0)),
        out_shape=jax.ShapeDtypeStruct((_NH, _D), jnp.float32),
        compiler_params=pltpu.CompilerParams(
            dimension_semantics=("parallel",)),
    )(g, g, g, w128)


@jax.jit
def _feat_propagation(parent_coord, s_coord, s_feat):
    p_pad = jnp.zeros((_N, 128), jnp.float32).at[:, :3].set(parent_coord)
    sx = jnp.zeros((128, _M), jnp.float32).at[:3, :].set(s_coord.T)
    sf_pad = jnp.zeros((_M, 128), jnp.float32).at[:, :_D].set(s_feat)
    outs = [_half_chain(p_pad[h * _NH:(h + 1) * _NH], sx, sf_pad)
            for h in range(_H)]
    return jnp.concatenate(outs, axis=0)


def kernel(parent_coord, parent_offset, s_coord, s_offset, s_feat):
    del parent_offset, s_offset
    return _feat_propagation(parent_coord, s_coord, s_feat)
